# baseline (speedup 1.0000x reference)
import functools

import jax
import jax.numpy as jnp
from jax import lax
from jax.experimental import pallas as pl
from jax.experimental.pallas import tpu as pltpu
from jax.experimental.pallas import tpu_sc as plsc

_N = 16384
_J = 50
_D = 64
_V = 1000000
_NC = 2
_NS = 16
_NW = _NC * _NS
_IPW = _N // _NW
_C = 128
_NT = _IPW // _C


def _emb_body(idx_hbm, table_hbm, out_hbm, idx_v, idx2_st, *scratch):
    gbufs = list(scratch[:_NT])
    tbufs = list(scratch[_NT:_NT + 2])
    gsems = list(scratch[_NT + 2:2 * _NT + 2])
    osems = list(scratch[2 * _NT + 2:2 * _NT + 4])

    wid = lax.axis_index("s") * _NC + lax.axis_index("c")
    ibase = wid * _IPW
    iota = lax.iota(jnp.int32, 16)

    pltpu.sync_copy(idx_hbm.at[:, pl.ds(wid * _NT, _NT), :], idx_v)

    def issue_gather(j, t, gb):
        for k in range(8):
            p = idx_v[j, t, pl.ds(k * 16, 16)]
            idx2_st[gb, pl.ds(k * 16, 16)] = p & 0x3FFFFFFF
        pltpu.make_async_copy(
            table_hbm.at[idx2_st.at[gb]], gbufs[gb], gsems[gb]).start()

    def wait_gather(gb):
        pltpu.make_async_copy(
            table_hbm.at[idx_v.at[0, 0]], gbufs[gb], gsems[gb]).wait()

    def issue_store(j, t, tb):
        pltpu.make_async_copy(
            tbufs[tb],
            out_hbm.at[j, :, pl.ds(ibase + t * _C, _C)],
            osems[tb]).start()

    def wait_store(tb):
        pltpu.make_async_copy(
            tbufs[tb], out_hbm.at[0, :, pl.ds(0, _C)], osems[tb]).wait()

    perm = [(iota + dd) & 15 for dd in range(16)]

    def transpose(gb, tb, j, t):
        gbuf, tbuf = gbufs[gb], tbufs[tb]

        @plsc.parallel_loop(0, 32, unroll=2)
        def sbody(s):
            kb16 = (s // 4) * 16
            d0 = (s % 4) * 16
            rowv = jnp.full((16,), 0, jnp.int32) + kb16 + iota
            hv64 = (idx_v[j, t, pl.ds(kb16, 16)] >> 30) * 64
            colbh = hv64 + d0
            rowp = rowv + d0 * 128
            for dd in range(16):
                v = plsc.load_gather(gbuf, [rowv, colbh + perm[dd]])
                plsc.store_scatter(tbuf, [perm[dd], rowp], v)

    def block(j, t, first, rearm):
        gb = t
        tb = t % 2
        wait_gather(gb)
        if not first:
            wait_store(tb)
        transpose(gb, tb, j, t)
        issue_store(j, t, tb)
        if rearm:
            issue_gather(j + 1, t, gb)

    for t in range(_NT):
        issue_gather(0, t, t)
    block(0, 0, True, True)
    block(0, 1, True, True)
    block(0, 2, False, True)
    block(0, 3, False, True)

    def outer(k, carry):
        for t in range(_NT):
            block(k, t, False, True)
        return carry

    lax.fori_loop(1, _J - 1, outer, 0)

    for t in range(_NT):
        block(_J - 1, t, False, False)
    wait_store(0)
    wait_store(1)


_RT = 1000000 // 128
_BPWP = _RT // _NW


def _prep_body(t2_hbm, tail_hbm, out_hbm, *scratch):
    ibufs = list(scratch[:4])
    tbufs = list(scratch[4:6])
    isems = list(scratch[6:10])
    osems = list(scratch[10:12])

    wid = lax.axis_index("s") * _NC + lax.axis_index("c")
    base = wid * _BPWP
    iota = lax.iota(jnp.int32, 16)
    perm = [(iota + dd) & 15 for dd in range(16)]

    def issue_read(i, ib):
        pltpu.make_async_copy(
            t2_hbm.at[:, pl.ds((base + i) * 128, 128)], ibufs[ib],
            isems[ib]).start()

    def wait_read(ib):
        pltpu.make_async_copy(
            t2_hbm.at[:, pl.ds(0, 128)], ibufs[ib], isems[ib]).wait()

    def issue_write(i, tb):
        pltpu.make_async_copy(
            tbufs[tb], out_hbm.at[pl.ds((base + i) * 64, 64), :],
            osems[tb]).start()

    def wait_write(tb):
        pltpu.make_async_copy(
            tbufs[tb], out_hbm.at[pl.ds(0, 64), :], osems[tb]).wait()

    qhalf = iota >> 1
    sel64 = (iota & 1) * 64

    def transpose(ib, tb):
        ibuf, tbuf = ibufs[ib], tbufs[tb]

        @plsc.parallel_loop(0, 32, unroll=2)
        def sbody(s):
            rb16 = (s // 4) * 16
            d0 = (s % 4) * 16
            rowv = jnp.full((16,), 0, jnp.int32) + rb16 + iota
            rowst = jnp.full((16,), 0, jnp.int32) + (s // 4) * 8 + qhalf
            rowp = rowv + d0 * 128
            colbs = sel64 + d0
            for dd in range(16):
                v = plsc.load_gather(ibuf, [perm[dd], rowp])
                plsc.store_scatter(tbuf, [rowst, colbs + perm[dd]], v)

    def blk(i, ib, tb, first, rearm):
        wait_read(ib)
        if not first:
            wait_write(tb)
        transpose(ib, tb)
        issue_write(i, tb)
        if rearm:
            issue_read(i + 4, ib)

    for t in range(4):
        issue_read(t, t)
    blk(0, 0, 0, True, True)
    blk(1, 1, 1, True, True)
    blk(2, 2, 0, False, True)
    blk(3, 3, 1, False, True)

    def outer(k, carry):
        for t in range(4):
            blk(k * 4 + t, t, t % 2, False, True)
        return carry

    lax.fori_loop(1, _BPWP // 4 - 1, outer, 0)

    for t in range(4):
        blk(_BPWP - 4 + t, t, t % 2, False, False)
    wait_write(0)
    wait_write(1)

    @pl.when(wid < 4)
    def _extra():
        rt = _NW * _BPWP + wid
        pltpu.sync_copy(t2_hbm.at[:, pl.ds(rt * 128, 128)], ibufs[0])
        transpose(0, 0)
        pltpu.sync_copy(tbufs[0], out_hbm.at[pl.ds(rt * 64, 64), :])

    @pl.when(wid == _NW - 1)
    def _tail():
        pltpu.sync_copy(tail_hbm, tbufs[1].at[pl.ds(0, 32), :])
        pltpu.sync_copy(
            tbufs[1].at[pl.ds(0, 32), :],
            out_hbm.at[pl.ds(_RT * 64, 32), :])


@jax.jit
def _prep(t2, tail_p):
    mesh = plsc.VectorSubcoreMesh(core_axis_name="c", subcore_axis_name="s")
    f = functools.partial(
        pl.kernel,
        mesh=mesh,
        out_type=jax.ShapeDtypeStruct((_V // 2, 128), jnp.float32),
        scratch_types=(
            [pltpu.VMEM((_D, 128), jnp.float32) for _ in range(4)]
            + [pltpu.VMEM((64, 128), jnp.float32) for _ in range(2)]
            + [pltpu.SemaphoreType.DMA for _ in range(6)]
        ),
        compiler_params=pltpu.CompilerParams(use_tc_tiling_on_sc=True,
                                               needs_layout_passes=False),
    )(_prep_body)
    return f(t2, tail_p)


@jax.jit
def _emb(idx, table):
    mesh = plsc.VectorSubcoreMesh(core_axis_name="c", subcore_axis_name="s")
    f = functools.partial(
        pl.kernel,
        mesh=mesh,
        out_type=jax.ShapeDtypeStruct((_J, _D, _N), jnp.float32),
        scratch_types=(
            [pltpu.VMEM((_J, _NT, _C), jnp.int32)]
            + [pltpu.VMEM((_NT, _C), jnp.int32)]
            + [pltpu.VMEM((_C, 128), jnp.float32) for _ in range(_NT)]
            + [pltpu.VMEM((_D, _C), jnp.float32) for _ in range(2)]
            + [pltpu.SemaphoreType.DMA for _ in range(_NT + 2)]
        ),
        compiler_params=pltpu.CompilerParams(use_tc_tiling_on_sc=True,
                                               needs_layout_passes=False),
    )(_emb_body)
    return f(idx, table)


def kernel(event_ids, event_embeddings_weight):
    r = event_ids.astype(jnp.int32)
    idx = ((r >> 1) | ((r & 1) << 30)).T.reshape(_J, _N // _C, _C)
    t2 = event_embeddings_weight.T
    tail_c = event_embeddings_weight[_RT * 128:].reshape(32, 128)
    fmt = _prep(t2, tail_c)
    out = _emb(idx, fmt)
    return jnp.transpose(out, (2, 0, 1))


# speedup vs baseline: 1.1120x; 1.1120x over previous
import functools

import jax
import jax.numpy as jnp
from jax import lax
from jax.experimental import pallas as pl
from jax.experimental.pallas import tpu as pltpu
from jax.experimental.pallas import tpu_sc as plsc

_N = 16384
_J = 50
_D = 64
_V = 1000000
_NC = 2
_NS = 16
_NW = _NC * _NS
_IPW = _N // _NW
_C = 128
_NT = _IPW // _C


def _emb_body(idx_hbm, table_hbm, out_hbm, idx_v, idx2_st, *scratch):
    gbufs = list(scratch[:_NT])
    tbufs = list(scratch[_NT:_NT + 2])
    gsems = list(scratch[_NT + 2:2 * _NT + 2])
    osems = list(scratch[2 * _NT + 2:2 * _NT + 4])

    wid = lax.axis_index("s") * _NC + lax.axis_index("c")
    ibase = wid * _IPW
    iota = lax.iota(jnp.int32, 16)

    pltpu.sync_copy(idx_hbm.at[:, pl.ds(wid * _NT, _NT), :], idx_v)

    def issue_gather(j, t, gb):
        for k in range(8):
            p = idx_v[j, t, pl.ds(k * 16, 16)]
            idx2_st[gb, pl.ds(k * 16, 16)] = p & 0x3FFFFFFF
        pltpu.make_async_copy(
            table_hbm.at[idx2_st.at[gb]], gbufs[gb], gsems[gb]).start()

    def wait_gather(gb):
        pltpu.make_async_copy(
            table_hbm.at[idx_v.at[0, 0]], gbufs[gb], gsems[gb]).wait()

    def issue_store(j, t, tb):
        pltpu.make_async_copy(
            tbufs[tb],
            out_hbm.at[j, :, pl.ds(ibase + t * _C, _C)],
            osems[tb]).start()

    def wait_store(tb):
        pltpu.make_async_copy(
            tbufs[tb], out_hbm.at[0, :, pl.ds(0, _C)], osems[tb]).wait()

    perm = [(iota + dd) & 15 for dd in range(16)]

    def transpose(gb, tb, j, t):
        gbuf, tbuf = gbufs[gb], tbufs[tb]

        @plsc.parallel_loop(0, 32, unroll=2)
        def sbody(s):
            kb16 = (s // 4) * 16
            d0 = (s % 4) * 16
            rowv = jnp.full((16,), 0, jnp.int32) + kb16 + iota
            colb = jnp.full((16,), 0, jnp.int32) + d0
            hv64 = (idx_v[j, t, pl.ds(kb16, 16)] >> 30) * 64
            for dd in range(16):
                colv = colb + perm[dd]
                v = plsc.load_gather(gbuf, [rowv, colv + hv64])
                plsc.store_scatter(tbuf, [colv, rowv], v)

    def block(j, t, first, rearm):
        gb = t
        tb = t % 2
        wait_gather(gb)
        if not first:
            wait_store(tb)
        transpose(gb, tb, j, t)
        issue_store(j, t, tb)
        if rearm:
            issue_gather(j + 1, t, gb)

    for t in range(_NT):
        issue_gather(0, t, t)
    block(0, 0, True, True)
    block(0, 1, True, True)
    block(0, 2, False, True)
    block(0, 3, False, True)

    def outer(k, carry):
        for t in range(_NT):
            block(k, t, False, True)
        return carry

    lax.fori_loop(1, _J - 1, outer, 0)

    for t in range(_NT):
        block(_J - 1, t, False, False)
    wait_store(0)
    wait_store(1)


_RT = 1000000 // 128
_BPWP = _RT // _NW


def _prep_body(t2_hbm, tail_hbm, out_hbm, *scratch):
    ibufs = list(scratch[:4])
    tbufs = list(scratch[4:6])
    isems = list(scratch[6:10])
    osems = list(scratch[10:12])

    wid = lax.axis_index("s") * _NC + lax.axis_index("c")
    base = wid * _BPWP
    iota = lax.iota(jnp.int32, 16)
    perm = [(iota + dd) & 15 for dd in range(16)]

    def issue_read(i, ib):
        pltpu.make_async_copy(
            t2_hbm.at[:, pl.ds((base + i) * 128, 128)], ibufs[ib],
            isems[ib]).start()

    def wait_read(ib):
        pltpu.make_async_copy(
            t2_hbm.at[:, pl.ds(0, 128)], ibufs[ib], isems[ib]).wait()

    def issue_write(i, tb):
        pltpu.make_async_copy(
            tbufs[tb], out_hbm.at[pl.ds((base + i) * 64, 64), :],
            osems[tb]).start()

    def wait_write(tb):
        pltpu.make_async_copy(
            tbufs[tb], out_hbm.at[pl.ds(0, 64), :], osems[tb]).wait()

    qhalf = iota >> 1
    sel64 = (iota & 1) * 64

    def transpose(ib, tb):
        ibuf, tbuf = ibufs[ib], tbufs[tb]

        @plsc.parallel_loop(0, 32, unroll=2)
        def sbody(s):
            rb16 = (s // 4) * 16
            d0 = (s % 4) * 16
            rowv = jnp.full((16,), 0, jnp.int32) + rb16 + iota
            rowst = jnp.full((16,), 0, jnp.int32) + (s // 4) * 8 + qhalf
            colb = jnp.full((16,), 0, jnp.int32) + d0
            for dd in range(16):
                colv = colb + perm[dd]
                v = plsc.load_gather(ibuf, [colv, rowv])
                plsc.store_scatter(tbuf, [rowst, colv + sel64], v)

    def blk(i, ib, tb, first, rearm):
        wait_read(ib)
        if not first:
            wait_write(tb)
        transpose(ib, tb)
        issue_write(i, tb)
        if rearm:
            issue_read(i + 4, ib)

    for t in range(4):
        issue_read(t, t)
    blk(0, 0, 0, True, True)
    blk(1, 1, 1, True, True)
    blk(2, 2, 0, False, True)
    blk(3, 3, 1, False, True)

    def outer(k, carry):
        for t in range(4):
            blk(k * 4 + t, t, t % 2, False, True)
        return carry

    lax.fori_loop(1, _BPWP // 4 - 1, outer, 0)

    for t in range(4):
        blk(_BPWP - 4 + t, t, t % 2, False, False)
    wait_write(0)
    wait_write(1)

    @pl.when(wid < 4)
    def _extra():
        rt = _NW * _BPWP + wid
        pltpu.sync_copy(t2_hbm.at[:, pl.ds(rt * 128, 128)], ibufs[0])
        transpose(0, 0)
        pltpu.sync_copy(tbufs[0], out_hbm.at[pl.ds(rt * 64, 64), :])

    @pl.when(wid == _NW - 1)
    def _tail():
        pltpu.sync_copy(tail_hbm, tbufs[1].at[pl.ds(0, 32), :])
        pltpu.sync_copy(
            tbufs[1].at[pl.ds(0, 32), :],
            out_hbm.at[pl.ds(_RT * 64, 32), :])


@jax.jit
def _prep(t2, tail_p):
    mesh = plsc.VectorSubcoreMesh(core_axis_name="c", subcore_axis_name="s")
    f = functools.partial(
        pl.kernel,
        mesh=mesh,
        out_type=jax.ShapeDtypeStruct((_V // 2, 128), jnp.float32),
        scratch_types=(
            [pltpu.VMEM((_D, 128), jnp.float32) for _ in range(4)]
            + [pltpu.VMEM((64, 128), jnp.float32) for _ in range(2)]
            + [pltpu.SemaphoreType.DMA for _ in range(6)]
        ),
        compiler_params=pltpu.CompilerParams(use_tc_tiling_on_sc=True,
                                               needs_layout_passes=False),
    )(_prep_body)
    return f(t2, tail_p)


@jax.jit
def _emb(idx, table):
    mesh = plsc.VectorSubcoreMesh(core_axis_name="c", subcore_axis_name="s")
    f = functools.partial(
        pl.kernel,
        mesh=mesh,
        out_type=jax.ShapeDtypeStruct((_J, _D, _N), jnp.float32),
        scratch_types=(
            [pltpu.VMEM((_J, _NT, _C), jnp.int32)]
            + [pltpu.VMEM((_NT, _C), jnp.int32)]
            + [pltpu.VMEM((_C, 128), jnp.float32) for _ in range(_NT)]
            + [pltpu.VMEM((_D, _C), jnp.float32) for _ in range(2)]
            + [pltpu.SemaphoreType.DMA for _ in range(_NT + 2)]
        ),
        compiler_params=pltpu.CompilerParams(use_tc_tiling_on_sc=True,
                                               needs_layout_passes=False),
    )(_emb_body)
    return f(idx, table)


def kernel(event_ids, event_embeddings_weight):
    r = event_ids.astype(jnp.int32)
    idx = ((r >> 1) | ((r & 1) << 30)).T.reshape(_J, _N // _C, _C)
    t2 = event_embeddings_weight.T
    tail_c = event_embeddings_weight[_RT * 128:].reshape(32, 128)
    fmt = _prep(t2, tail_c)
    out = _emb(idx, fmt)
    return jnp.transpose(out, (2, 0, 1))
